# Initial kernel scaffold; baseline (speedup 1.0000x reference)
import jax
import jax.numpy as jnp
from jax.experimental import pallas as pl


def kernel(features, edge_index, segment, Max_atoms, T, equation, num_mols, training, W_g0, b_g0, W_g1, b_g1, W_t, b_t, W0, b0, W1, b1, W2, b2, W3, b3, W4, b4, W5, b5, W6, b6, W7, b7, W_r0, b_r0, W_r1, b_r1, W_r2, b_r2):
    raise NotImplementedError("write your pallas kernel here")


# speedup vs baseline: 3.3498x; 3.3498x over previous
import functools

import jax
import jax.numpy as jnp
from jax import lax
from jax.experimental import pallas as pl
from jax.experimental.pallas import tpu as pltpu
from jax.experimental.pallas import tpu_sc as plsc

_N = 10000
_D = 128
_NPAD = 10240
_NROW = _NPAD // 128
_E = 320000
_NC = 2
_NS = 16
_NW = _NC * _NS
_EPU = 128
_UPW = 79
_EPAD = _NW * _UPW * _EPU
_EU = _EPAD // _EPU


def _sc_mesh():
    return plsc.VectorSubcoreMesh(
        core_axis_name="c", subcore_axis_name="s",
        num_cores=_NC, num_subcores=_NS)


_DW = 16


def _sc_degrees(src1d, dst1d, ones_blk, zeros_nd):
    @functools.partial(
        pl.kernel,
        out_type=jax.ShapeDtypeStruct((_NC, 2, _NPAD, 128), jnp.float32),
        mesh=_sc_mesh(),
        scratch_types=[
            pltpu.VMEM((128,), jnp.int32),
            pltpu.VMEM((128, 128), jnp.float32),
            pltpu.VMEM_SHARED((_NPAD, 128), jnp.float32),
        ],
    )
    def deg_kernel(src_hbm, dst_hbm, ones_hbm, zeros_hbm, out_hbm,
                   sidx, ones_v, acc):
        c = lax.axis_index("c")
        s = lax.axis_index("s")
        wid = s * _NC + c
        nrow_t = _NPAD // _NS
        sl = pl.ds(s * nrow_t, nrow_t)
        pltpu.sync_copy(ones_hbm, ones_v)
        for phase, idx_hbm in ((0, src_hbm), (1, dst_hbm)):
            pltpu.sync_copy(zeros_hbm.at[sl], acc.at[sl])
            plsc.subcore_barrier()

            def unit(u, carry):
                e0 = (wid * _UPW + u) * _EPU
                pltpu.sync_copy(idx_hbm.at[pl.ds(e0, _EPU)], sidx)
                pltpu.sync_copy(ones_v, acc.at[sidx], add=True)
                return carry
            lax.fori_loop(0, _UPW, unit, 0)

            plsc.subcore_barrier()
            pltpu.sync_copy(acc.at[sl], out_hbm.at[c, phase, sl])
            plsc.subcore_barrier()

    return deg_kernel(src1d, dst1d, ones_blk, zeros_nd)


def _sc_msg(table, src1d, dst1d, zeros_nd):
    @functools.partial(
        pl.kernel,
        out_type=jax.ShapeDtypeStruct((_NC, _NPAD, 128), jnp.float32),
        mesh=_sc_mesh(),
        scratch_types=[
            pltpu.VMEM((128,), jnp.int32),
            pltpu.VMEM((128,), jnp.int32),
            pltpu.VMEM((128, 128), jnp.float32),
            pltpu.VMEM_SHARED((_NPAD, 128), jnp.float32),
            pltpu.SemaphoreType.DMA,
        ],
    )
    def msg_kernel(table_hbm, src_hbm, dst_hbm, zeros_hbm, out_hbm,
                   sidx, didx, rows, acc, sem):
        c = lax.axis_index("c")
        s = lax.axis_index("s")
        wid = s * _NC + c
        nrow_t = _NPAD // _NS

        pltpu.sync_copy(zeros_hbm.at[pl.ds(s * nrow_t, nrow_t)],
                        acc.at[pl.ds(s * nrow_t, nrow_t)])
        plsc.subcore_barrier()

        def unit(u, carry):
            e0 = (wid * _UPW + u) * _EPU
            pltpu.sync_copy(src_hbm.at[pl.ds(e0, _EPU)], sidx)
            pltpu.async_copy(table_hbm.at[sidx], rows, sem).wait()
            pltpu.sync_copy(dst_hbm.at[pl.ds(e0, _EPU)], didx)
            pltpu.sync_copy(rows, acc.at[didx], add=True)
            return carry
        lax.fori_loop(0, _UPW, unit, 0)

        plsc.subcore_barrier()
        pltpu.sync_copy(acc.at[pl.ds(s * nrow_t, nrow_t)],
                        out_hbm.at[c, pl.ds(s * nrow_t, nrow_t)])

    return msg_kernel(table, src1d, dst1d, zeros_nd)


def _scales(deg2d):
    s_out = lax.rsqrt(jnp.maximum(deg2d[0], 1.0))
    s_in = lax.rsqrt(jnp.maximum(deg2d[1], 1.0))
    return s_out, s_in


def _tc_prep(fpad, deg_parts):
    def body(f_ref, dp_ref, out_ref):
        s_out, _ = _scales(dp_ref[...])
        f3 = f_ref[...].reshape(_NROW, 128, 128)
        out_ref[...] = (f3 * s_out[:, :, None]).reshape(_NPAD, 128)
    return pl.pallas_call(
        body,
        out_shape=jax.ShapeDtypeStruct((_NPAD, 128), jnp.float32),
    )(fpad, deg_parts)


def _tc_layer(mp, deg_parts, W, b):
    def body(mp_ref, dp_ref, w_ref, b_ref, out_ref):
        s_out, s_in = _scales(dp_ref[...])
        mm = mp_ref[...]
        m3 = (mm[0] + mm[1]).reshape(_NROW, 128, 128) * s_in[:, :, None]
        h = jnp.dot(m3.reshape(_NPAD, 128), w_ref[...],
                    preferred_element_type=jnp.float32) + b_ref[...]
        h = jnp.maximum(h, 0.0)
        out_ref[...] = (h.reshape(_NROW, 128, 128)
                        * s_out[:, :, None]).reshape(_NPAD, 128)
    return pl.pallas_call(
        body,
        out_shape=jax.ShapeDtypeStruct((_NPAD, 128), jnp.float32),
    )(mp, deg_parts, W, b.reshape(1, -1))


def _tc_head(mp, deg_parts, Wg, bg, T2, Wt, bt,
             W0, b0, W1, b1, W2, b2, W3, b3, W4, b4, W5, b5, W6, b6, W7, b7,
             W_r0, b_r0, W_r1, b_r1, W_r2p, b_r2p, n_groups, n_mols, max_atoms):
    def body(mp_ref, dp_ref, wg_ref, bg_ref, t_ref, wt_ref, bt_ref,
             w0_ref, b0_ref, w1_ref, b1_ref, w2_ref, b2_ref, w3_ref, b3_ref,
             w4_ref, b4_ref, w5_ref, b5_ref, w6_ref, b6_ref, w7_ref, b7_ref,
             wr0_ref, br0_ref, wr1_ref, br1_ref, wr2_ref, br2_ref, out_ref):
        _, s_in = _scales(dp_ref[...])
        mm = mp_ref[...]
        m3 = (mm[0] + mm[1]).reshape(_NROW, 128, 128) * s_in[:, :, None]
        h2 = jnp.dot(m3.reshape(_NPAD, 128), wg_ref[...],
                     preferred_element_type=jnp.float32) + bg_ref[...]
        h2 = jnp.maximum(h2, 0.0)[:_N]
        u1 = jnp.dot(jnp.dot(h2, w0_ref[...],
                             preferred_element_type=jnp.float32) + b0_ref[...],
                     w1_ref[...], preferred_element_type=jnp.float32) + b1_ref[...]
        h2g = h2.reshape(n_groups, max_atoms, 128)
        hmean = jnp.mean(h2g, axis=1)
        T_part = t_ref[...] * wt_ref[...] + bt_ref[...]
        t2 = jnp.dot(jnp.dot(T_part, w2_ref[...],
                             preferred_element_type=jnp.float32) + b2_ref[...],
                     w3_ref[...], preferred_element_type=jnp.float32) + b3_ref[...]
        u1g = u1.reshape(n_groups, max_atoms, 128)
        t2g = t2.reshape(n_groups, n_mols, 128)
        cols = []
        for m_ in range(n_mols):
            t2m = t2g[:, m_, :][:, None, :]
            sm = jnp.sum(jnp.maximum(u1g + t2m, 0.0), axis=1)
            cols.append(hmean + sm * (1.0 / max_atoms))
        mean = jnp.stack(cols, axis=1).reshape(-1, 128)
        update3 = jnp.dot(jnp.dot(mean, w4_ref[...],
                                  preferred_element_type=jnp.float32) + b4_ref[...],
                          w5_ref[...], preferred_element_type=jnp.float32) + b5_ref[...]
        update4 = jnp.dot(jnp.dot(T_part, w6_ref[...],
                                  preferred_element_type=jnp.float32) + b6_ref[...],
                          w7_ref[...], preferred_element_type=jnp.float32) + b7_ref[...]
        updated_T = T_part + jnp.maximum(update3 + update4, 0.0)
        cv = jnp.concatenate([updated_T, mean], axis=-1)
        hh = jnp.maximum(jnp.dot(cv, wr0_ref[...],
                                 preferred_element_type=jnp.float32) + br0_ref[...], 0.0)
        hh = jnp.maximum(jnp.dot(hh, wr1_ref[...],
                                 preferred_element_type=jnp.float32) + br1_ref[...], 0.0)
        out_ref[...] = jnp.maximum(
            jnp.dot(hh, wr2_ref[...],
                    preferred_element_type=jnp.float32) + br2_ref[...], 0.0)
    nb = T2.shape[0]
    return pl.pallas_call(
        body,
        out_shape=jax.ShapeDtypeStruct((nb, 128), jnp.float32),
    )(mp, deg_parts, Wg, bg.reshape(1, -1), T2, Wt.reshape(1, -1),
      bt.reshape(1, -1),
      W0, b0.reshape(1, -1), W1, b1.reshape(1, -1), W2, b2.reshape(1, -1),
      W3, b3.reshape(1, -1), W4, b4.reshape(1, -1), W5, b5.reshape(1, -1),
      W6, b6.reshape(1, -1), W7, b7.reshape(1, -1),
      W_r0, b_r0.reshape(1, -1), W_r1, b_r1.reshape(1, -1), W_r2p, b_r2p)


def kernel(features, edge_index, segment, Max_atoms, T, equation, num_mols,
           training, W_g0, b_g0, W_g1, b_g1, W_t, b_t,
           W0, b0, W1, b1, W2, b2, W3, b3, W4, b4, W5, b5, W6, b6, W7, b7,
           W_r0, b_r0, W_r1, b_r1, W_r2, b_r2):
    n = features.shape[0]
    max_atoms = segment.shape[0] // T.shape[0]
    n_mols = T.shape[0] // (n // max_atoms)
    n_groups = n // max_atoms

    src = edge_index[0]
    dst = edge_index[1]
    pad = _EPAD - _E
    srcp = jnp.concatenate([src, jnp.full((pad,), _N, jnp.int32)])
    dstp = jnp.concatenate([dst, jnp.full((pad,), _N, jnp.int32)])
    fpad = jnp.pad(features, ((0, _NPAD - n), (0, 0)))
    zeros_nd = jnp.zeros((_NPAD, _D), jnp.float32)

    ones_blk = jnp.ones((128, 128), jnp.float32)
    deg_parts = _sc_degrees(srcp, dstp, ones_blk, zeros_nd)
    deg2d = (deg_parts[0] + deg_parts[1])[:, :, 0].reshape(2, _NROW, 128)
    h0s = _tc_prep(fpad, deg2d)
    m1p = _sc_msg(h0s, srcp, dstp, zeros_nd)
    h1s = _tc_layer(m1p, deg2d, W_g0, b_g0)
    m2p = _sc_msg(h1s, srcp, dstp, zeros_nd)

    W_r2p = jnp.zeros((128, 128), jnp.float32).at[:, :2].set(W_r2)
    b_r2p = jnp.zeros((1, 128), jnp.float32).at[0, :2].set(b_r2)
    predp = _tc_head(m2p, deg2d, W_g1, b_g1, T.reshape(-1, 1), W_t, b_t,
                     W0, b0, W1, b1, W2, b2, W3, b3, W4, b4, W5, b5,
                     W6, b6, W7, b7, W_r0, b_r0, W_r1, b_r1, W_r2p, b_r2p,
                     n_groups, n_mols, max_atoms)
    return predp[:, :2]
